# baseline (speedup 1.0000x reference)
import functools

import jax
import jax.numpy as jnp
from jax import lax
from jax.experimental import pallas as pl
from jax.experimental.pallas import tpu as pltpu
from jax.experimental.pallas import tpu_sc as plsc

D = 64
B = 4096
NC, NS, L = 2, 16, 16
NW = NC * NS
RPW = B // NW
MAX_NORM = 1.0
EPS = 1e-7


def _permute(x, idx):
    dnums = lax.GatherDimensionNumbers(
        offset_dims=(), collapsed_slice_dims=(0,), start_index_map=(0,))
    return lax.gather(x, idx[:, None], dnums, slice_sizes=(1,),
                      mode=lax.GatherScatterMode.PROMISE_IN_BOUNDS)


def _rsqrt(x):
    i = plsc.bitcast(x, jnp.int32)
    i = jnp.int32(0x5F3759DF) - lax.shift_right_logical(i, 1)
    y = plsc.bitcast(i, jnp.float32)
    for _ in range(3):
        y = y * (1.5 - 0.5 * x * y * y)
    return y


def _scale_rows(rows):
    def body(r, carry):
        v0 = rows[r, pl.ds(0 * L, L)]
        v1 = rows[r, pl.ds(1 * L, L)]
        v2 = rows[r, pl.ds(2 * L, L)]
        v3 = rows[r, pl.ds(3 * L, L)]
        acc = v0 * v0 + v1 * v1 + v2 * v2 + v3 * v3
        lane = lax.iota(jnp.int32, L)
        for k in (1, 2, 4, 8):
            acc = acc + _permute(acc, lane ^ k)
        norm = acc * _rsqrt(acc)
        scale = jnp.where(acc > MAX_NORM * MAX_NORM,
                          MAX_NORM / (norm + EPS),
                          jnp.full((L,), 1.0, dtype=jnp.float32))
        rows[r, pl.ds(0 * L, L)] = v0 * scale
        rows[r, pl.ds(1 * L, L)] = v1 * scale
        rows[r, pl.ds(2 * L, L)] = v2 * scale
        rows[r, pl.ds(3 * L, L)] = v3 * scale
        return carry

    lax.fori_loop(0, RPW, body, 0)


@functools.partial(
    pl.kernel,
    out_type=(
        jax.ShapeDtypeStruct((B, D), jnp.float32),
        jax.ShapeDtypeStruct((B, D), jnp.float32),
        jax.ShapeDtypeStruct((B, D), jnp.float32),
    ),
    mesh=plsc.VectorSubcoreMesh(core_axis_name="c", subcore_axis_name="s"),
    compiler_params=pltpu.CompilerParams(needs_layout_passes=False,
                                         use_tc_tiling_on_sc=True),
    scratch_types=[
        pltpu.VMEM((RPW,), jnp.int32),
        pltpu.VMEM((RPW,), jnp.int32),
        pltpu.VMEM((RPW,), jnp.int32),
        pltpu.VMEM((RPW, D), jnp.float32),
        pltpu.VMEM((RPW, D), jnp.float32),
        pltpu.VMEM((RPW, D), jnp.float32),
        pltpu.SemaphoreType.DMA,
        pltpu.SemaphoreType.DMA,
        pltpu.SemaphoreType.DMA,
    ],
)
def _sc_lookup(ig, ia, ie, wg, wa, we, og, oa, oe,
               xg, xa, xe, rg, ra, re, sg, sa, se):
    wid = lax.axis_index("s") * NC + lax.axis_index("c")
    base = wid * RPW
    for idx_hbm, idx_v, table, rows_v, sem in (
            (ig, xg, wg, rg, sg), (ia, xa, wa, ra, sa), (ie, xe, we, re, se)):
        pltpu.sync_copy(idx_hbm.at[pl.ds(base, RPW)], idx_v)
        for c in range(RPW // L):
            ids = idx_v[pl.ds(c * L, L)]
            for l in range(L):
                pltpu.async_copy(table.at[pl.ds(ids[l], 1), :],
                                 rows_v.at[pl.ds(c * L + l, 1), :], sem)
    for idx_hbm, rows_v, table, out_hbm, sem in (
            (ig, rg, wg, og, sg), (ia, ra, wa, oa, sa), (ie, re, we, oe, se)):
        pltpu.make_async_copy(table.at[pl.ds(0, RPW), :], rows_v, sem).wait()
        _scale_rows(rows_v)
        pltpu.sync_copy(rows_v, out_hbm.at[pl.ds(base, RPW)])


def kernel(latent_idx_geo, latent_idx_app, latent_idx_exp, W_geo, W_app, W_exp):
    return _sc_lookup(latent_idx_geo.astype(jnp.int32),
                      latent_idx_app.astype(jnp.int32),
                      latent_idx_exp.astype(jnp.int32),
                      W_geo, W_app, W_exp)


# speedup vs baseline: 2.0650x; 2.0650x over previous
import functools

import jax
import jax.numpy as jnp
from jax import lax
from jax.experimental import pallas as pl
from jax.experimental.pallas import tpu as pltpu
from jax.experimental.pallas import tpu_sc as plsc

D = 64
B = 4096
NC, NS, L = 2, 16, 16
NW = NC * NS
RPW = B // NW
NRING = 4
MAX_NORM = 1.0
EPS = 1e-7


def _rsqrt(x):
    i = plsc.bitcast(x, jnp.int32)
    i = jnp.int32(0x5F3759DF) - lax.shift_right_logical(i, 1)
    y = plsc.bitcast(i, jnp.float32)
    for _ in range(3):
        y = y * (1.5 - 0.5 * x * y * y)
    return y


def _scale_cols(cols):
    def group(g, carry):
        def sumsq(f, acc):
            v = cols[f, pl.ds(g * L, L)]
            return acc + v * v

        acc = lax.fori_loop(0, D, sumsq, jnp.zeros((L,), jnp.float32))
        norm = acc * _rsqrt(acc)
        scale = jnp.where(acc > MAX_NORM * MAX_NORM,
                          MAX_NORM / (norm + EPS),
                          jnp.full((L,), 1.0, dtype=jnp.float32))

        def apply(f, carry2):
            cols[f, pl.ds(g * L, L)] = cols[f, pl.ds(g * L, L)] * scale
            return carry2

        lax.fori_loop(0, D, apply, 0)
        return carry

    lax.fori_loop(0, RPW // L, group, 0)


def _idx_at(idx_v, j):
    v = plsc.load_gather(idx_v, [jnp.full((L,), j, jnp.int32)])
    return v[0]


@functools.partial(
    pl.kernel,
    out_type=(
        jax.ShapeDtypeStruct((D, B), jnp.float32),
        jax.ShapeDtypeStruct((D, B), jnp.float32),
        jax.ShapeDtypeStruct((D, B), jnp.float32),
    ),
    mesh=plsc.VectorSubcoreMesh(core_axis_name="c", subcore_axis_name="s"),
    compiler_params=pltpu.CompilerParams(needs_layout_passes=False),
    scratch_types=[
        pltpu.VMEM((RPW,), jnp.int32),
        pltpu.VMEM((RPW,), jnp.int32),
        pltpu.VMEM((RPW,), jnp.int32),
        pltpu.VMEM((NRING, D, 128), jnp.float32),
        pltpu.VMEM((D, RPW), jnp.float32),
        pltpu.VMEM((RPW, 128), jnp.float32),
        pltpu.VMEM((RPW, 128), jnp.float32),
        pltpu.VMEM((D, RPW), jnp.float32),
        pltpu.VMEM((D, RPW), jnp.float32),
        pltpu.SemaphoreType.DMA,
        pltpu.SemaphoreType.DMA,
        pltpu.SemaphoreType.DMA,
        pltpu.SemaphoreType.DMA,
        pltpu.SemaphoreType.DMA,
        pltpu.SemaphoreType.DMA,
    ],
)
def _sc_lookup(ig, ia, ie, wgt, wa2, we2, og, oa, oe,
               xg, xa, xe, ring, gcols, arows, erows, acols, ecols,
               r0, r1, r2, r3, sa, se):
    rsems = (r0, r1, r2, r3)
    wid = lax.axis_index("s") * NC + lax.axis_index("c")
    base = wid * RPW
    lane = lax.iota(jnp.int32, L)

    pltpu.sync_copy(ig.at[pl.ds(base, RPW)], xg)
    pltpu.sync_copy(ia.at[pl.ds(base, RPW)], xa)
    pltpu.sync_copy(ie.at[pl.ds(base, RPW)], xe)

    for idx_v, table2, rows_v, sem in ((xa, wa2, arows, sa),
                                       (xe, we2, erows, se)):
        for c in range(RPW // L):
            ids = idx_v[pl.ds(c * L, L)]
            for l in range(L):
                pltpu.async_copy(
                    table2.at[pl.ds(lax.shift_right_logical(ids[l], 1), 1), :],
                    rows_v.at[pl.ds(c * L + l, 1), :], sem)

    def _fire(j, slot, sem):
        i = _idx_at(xg, j)
        blk = lax.shift_right_logical(i, 7) * 128
        pltpu.async_copy(wgt.at[:, pl.ds(blk, 128)], ring.at[slot], sem)

    for s in range(NRING):
        _fire(jnp.int32(s), s, rsems[s])

    def round_body(r, carry):
        for s in range(NRING):
            j = r * NRING + s
            pltpu.make_async_copy(wgt.at[:, pl.ds(0, 128)], ring.at[s],
                                  rsems[s]).wait()
            i = _idx_at(xg, j)
            c = i & 127
            for g in range(D // L):
                f_lane = g * L + lane
                v = plsc.load_gather(
                    ring, [jnp.full((L,), s, jnp.int32), f_lane,
                           jnp.full((L,), c, jnp.int32)])
                plsc.store_scatter(gcols, [f_lane, jnp.full((L,), j,
                                                            jnp.int32)], v)

            @pl.when(r < RPW // NRING - 1)
            def _():
                _fire(j + NRING, s, rsems[s])
        return carry

    lax.fori_loop(0, RPW // NRING, round_body, 0)
    _scale_cols(gcols)
    pltpu.sync_copy(gcols, og.at[:, pl.ds(base, RPW)])

    for idx_v, rows_v, cols_v, out_hbm, table2, sem in (
            (xa, arows, acols, oa, wa2, sa), (xe, erows, ecols, oe, we2, se)):
        pltpu.make_async_copy(table2.at[pl.ds(0, RPW), :], rows_v, sem).wait()
        for g in range(RPW // L):
            rids = g * L + lane
            offs = (idx_v[pl.ds(g * L, L)] & 1) * D

            def tr_body(f, carry, rids=rids, offs=offs, rows_v=rows_v,
                        cols_v=cols_v, g=g):
                v = plsc.load_gather(rows_v, [rids, offs + f])
                cols_v[f, pl.ds(g * L, L)] = v
                return carry

            lax.fori_loop(0, D, tr_body, 0)
        _scale_cols(cols_v)
        pltpu.sync_copy(cols_v, out_hbm.at[:, pl.ds(base, RPW)])


def kernel(latent_idx_geo, latent_idx_app, latent_idx_exp, W_geo, W_app, W_exp):
    og, oa, oe = _sc_lookup(latent_idx_geo.astype(jnp.int32),
                            latent_idx_app.astype(jnp.int32),
                            latent_idx_exp.astype(jnp.int32),
                            W_geo.T,
                            W_app.reshape(-1, 2 * D),
                            W_exp.reshape(-1, 2 * D))
    return (og.T, oa.T, oe.T)


# speedup vs baseline: 2.1520x; 1.0421x over previous
import functools

import jax
import jax.numpy as jnp
from jax import lax
from jax.experimental import pallas as pl
from jax.experimental.pallas import tpu as pltpu
from jax.experimental.pallas import tpu_sc as plsc

D = 64
B = 4096
NC, NS, L = 2, 16, 16
NW = NC * NS
RPW = B // NW
NRING = 8
MAX_NORM = 1.0
EPS = 1e-7


def _rsqrt(x):
    i = plsc.bitcast(x, jnp.int32)
    i = jnp.int32(0x5F3759DF) - lax.shift_right_logical(i, 1)
    y = plsc.bitcast(i, jnp.float32)
    for _ in range(3):
        y = y * (1.5 - 0.5 * x * y * y)
    return y


def _scale_cols(cols):
    def group(g, carry):
        def sumsq(f, acc):
            v = cols[f, pl.ds(g * L, L)]
            return acc + v * v

        acc = lax.fori_loop(0, D, sumsq, jnp.zeros((L,), jnp.float32))
        norm = acc * _rsqrt(acc)
        scale = jnp.where(acc > MAX_NORM * MAX_NORM,
                          MAX_NORM / (norm + EPS),
                          jnp.full((L,), 1.0, dtype=jnp.float32))

        def apply(f, carry2):
            cols[f, pl.ds(g * L, L)] = cols[f, pl.ds(g * L, L)] * scale
            return carry2

        lax.fori_loop(0, D, apply, 0)
        return carry

    lax.fori_loop(0, RPW // L, group, 0)


def _idx_at(idx_v, j):
    v = plsc.load_gather(idx_v, [jnp.full((L,), j, jnp.int32)])
    return v[0]


@functools.partial(
    pl.kernel,
    out_type=(
        jax.ShapeDtypeStruct((D, B), jnp.float32),
        jax.ShapeDtypeStruct((D, B), jnp.float32),
        jax.ShapeDtypeStruct((D, B), jnp.float32),
    ),
    mesh=plsc.VectorSubcoreMesh(core_axis_name="c", subcore_axis_name="s"),
    compiler_params=pltpu.CompilerParams(needs_layout_passes=False),
    scratch_types=[
        pltpu.VMEM((RPW,), jnp.int32),
        pltpu.VMEM((RPW,), jnp.int32),
        pltpu.VMEM((RPW,), jnp.int32),
        pltpu.VMEM((NRING, D, 128), jnp.float32),
        pltpu.VMEM((D, RPW), jnp.float32),
        pltpu.VMEM((RPW, 128), jnp.float32),
        pltpu.VMEM((RPW, 128), jnp.float32),
        pltpu.VMEM((D, RPW), jnp.float32),
        pltpu.VMEM((D, RPW), jnp.float32),
        pltpu.SemaphoreType.DMA,
        pltpu.SemaphoreType.DMA,
        pltpu.SemaphoreType.DMA,
        pltpu.SemaphoreType.DMA,
        pltpu.SemaphoreType.DMA,
        pltpu.SemaphoreType.DMA,
        pltpu.SemaphoreType.DMA,
        pltpu.SemaphoreType.DMA,
        pltpu.SemaphoreType.DMA,
        pltpu.SemaphoreType.DMA,
    ],
)
def _sc_lookup(ig, ia, ie, wgt, wa2, we2, og, oa, oe,
               xg, xa, xe, ring, gcols, arows, erows, acols, ecols,
               r0, r1, r2, r3, r4, r5, r6, r7, sa, se):
    rsems = (r0, r1, r2, r3, r4, r5, r6, r7)
    wid = lax.axis_index("s") * NC + lax.axis_index("c")
    base = wid * RPW
    lane = lax.iota(jnp.int32, L)

    pltpu.sync_copy(ig.at[pl.ds(base, RPW)], xg)
    pltpu.sync_copy(ia.at[pl.ds(base, RPW)], xa)
    pltpu.sync_copy(ie.at[pl.ds(base, RPW)], xe)

    def _fire(j, slot, sem):
        i = _idx_at(xg, j)
        blk = lax.shift_right_logical(i, 7) * 128
        pltpu.async_copy(wgt.at[:, pl.ds(blk, 128)], ring.at[slot], sem)

    for s in range(NRING):
        _fire(jnp.int32(s), s, rsems[s])

    for idx_v, table2, rows_v, sem in ((xa, wa2, arows, sa),
                                       (xe, we2, erows, se)):
        for c in range(RPW // L):
            ids = idx_v[pl.ds(c * L, L)]
            for l in range(L):
                pltpu.async_copy(
                    table2.at[pl.ds(lax.shift_right_logical(ids[l], 1), 1), :],
                    rows_v.at[pl.ds(c * L + l, 1), :], sem)

    def round_body(r, carry):
        for s in range(NRING):
            j = r * NRING + s
            pltpu.make_async_copy(wgt.at[:, pl.ds(0, 128)], ring.at[s],
                                  rsems[s]).wait()
            i = _idx_at(xg, j)
            c = i & 127
            for g in range(D // L):
                f_lane = g * L + lane
                v = plsc.load_gather(
                    ring, [jnp.full((L,), s, jnp.int32), f_lane,
                           jnp.full((L,), c, jnp.int32)])
                plsc.store_scatter(gcols, [f_lane, jnp.full((L,), j,
                                                            jnp.int32)], v)

            @pl.when(r < RPW // NRING - 1)
            def _():
                _fire(j + NRING, s, rsems[s])
        return carry

    lax.fori_loop(0, RPW // NRING, round_body, 0)
    _scale_cols(gcols)
    pltpu.sync_copy(gcols, og.at[:, pl.ds(base, RPW)])

    for idx_v, rows_v, cols_v, out_hbm, table2, sem in (
            (xa, arows, acols, oa, wa2, sa), (xe, erows, ecols, oe, we2, se)):
        pltpu.make_async_copy(table2.at[pl.ds(0, RPW), :], rows_v, sem).wait()
        for g in range(RPW // L):
            rids = g * L + lane
            offs = (idx_v[pl.ds(g * L, L)] & 1) * D

            def tr_body(f, carry, rids=rids, offs=offs, rows_v=rows_v,
                        cols_v=cols_v, g=g):
                v = plsc.load_gather(rows_v, [rids, offs + f])
                cols_v[f, pl.ds(g * L, L)] = v
                return carry

            lax.fori_loop(0, D, tr_body, 0)
        _scale_cols(cols_v)
        pltpu.sync_copy(cols_v, out_hbm.at[:, pl.ds(base, RPW)])


def kernel(latent_idx_geo, latent_idx_app, latent_idx_exp, W_geo, W_app, W_exp):
    og, oa, oe = _sc_lookup(latent_idx_geo.astype(jnp.int32),
                            latent_idx_app.astype(jnp.int32),
                            latent_idx_exp.astype(jnp.int32),
                            W_geo.T,
                            W_app.reshape(-1, 2 * D),
                            W_exp.reshape(-1, 2 * D))
    return (og.T, oa.T, oe.T)
